# Initial kernel scaffold; baseline (speedup 1.0000x reference)
import jax
import jax.numpy as jnp
from jax.experimental import pallas as pl


def kernel(obs, act_p, rew_p, done, time_init, act_table, time_table, rew_W, rew_b):
    raise NotImplementedError("write your pallas kernel here")


# speedup vs baseline: 1.0958x; 1.0958x over previous
import functools

import jax
import jax.numpy as jnp
from jax import lax
from jax.experimental import pallas as pl
from jax.experimental.pallas import tpu as pltpu
from jax.experimental.pallas import tpu_sc as plsc

D = 128
T = 8192
NC = 2
NS = 16
L = 16
NW = NC * NS
CPW = T // NW
SUB = 128
NSUB = CPW // SUB


def _sc_body(obs_h, actp_h, rew_h, done_h, t0_h, actT_h, timeT_h, w_h, b_h,
             x_h, tout_h,
             done_v, tidx_v, aidx_v, obs_v, trow_v, arow_v,
             rew_v, w_v, b_v, t0_v, tout_v,
             sem0, sem1, sem2):
    c = lax.axis_index("c")
    s = lax.axis_index("s")
    wid = s * NC + c
    base = wid * CPW

    pltpu.sync_copy(done_h, done_v)
    pltpu.sync_copy(w_h, w_v)
    pltpu.sync_copy(b_h, b_v)
    pltpu.sync_copy(t0_h, t0_v)
    t0 = t0_v[...]
    lanes = lax.broadcasted_iota(jnp.int32, (L,), 0)

    def pref(k, mv):
        off = k * L
        dv = done_v[pl.ds(off, L)]
        iv = lanes + off + t0
        return jnp.maximum(mv, iv * dv)

    mv = lax.fori_loop(0, wid * (CPW // L), pref, jnp.zeros((L,), jnp.int32))
    m = jnp.max(mv)

    for kk in range(CPW // L):
        off = base + kk * L
        dv = done_v[pl.ds(off, L)]
        iv = lanes + off + t0
        vals = iv * dv
        eff = jnp.maximum(plsc.cummax(vals), jnp.full((L,), m, jnp.int32))
        tidx_v[kk // (SUB // L), pl.ds((kk % (SUB // L)) * L, L)] = iv - eff
        m = jnp.max(eff)

    for sub in range(NSUB):
        sb = base + sub * SUB
        pltpu.sync_copy(actp_h.at[pl.ds(sb, SUB)], aidx_v.at[sub])
        pltpu.sync_copy(rew_h.at[pl.ds(sb, SUB)], rew_v)
        cp0 = pltpu.async_copy(obs_h.at[pl.ds(sb, SUB)], obs_v, sem0)
        cp1 = pltpu.async_copy(timeT_h.at[tidx_v.at[sub]], trow_v, sem1)
        cp2 = pltpu.async_copy(actT_h.at[aidx_v.at[sub]], arow_v, sem2)
        cp0.wait()
        cp1.wait()
        cp2.wait()

        def row(r, _):
            rs = plsc.load_gather(rew_v, [jnp.full((L,), r, jnp.int32)])
            for j in range(D // L):
                sl = pl.ds(j * L, L)
                obs_v[r, sl] = (obs_v[r, sl] + trow_v[r, sl] + arow_v[r, sl]
                                + rs * w_v[sl] + b_v[sl])
            return 0

        lax.fori_loop(0, SUB, row, 0)
        pltpu.sync_copy(obs_v, x_h.at[pl.ds(sb, SUB)])

    @pl.when(wid == NW - 1)
    def _():
        tout_v[...] = t0 + (jnp.int32(T) - m)
        pltpu.sync_copy(tout_v, tout_h)


@jax.jit
def _run(obs, act_i, rew_p, done_i, t0_vec, act_table, time_table, w, b):
    mesh = plsc.VectorSubcoreMesh(core_axis_name="c", subcore_axis_name="s")
    f = functools.partial(
        pl.kernel,
        out_type=[
            jax.ShapeDtypeStruct((T, D), jnp.float32),
            jax.ShapeDtypeStruct((L,), jnp.int32),
        ],
        mesh=mesh,
        compiler_params=pltpu.CompilerParams(needs_layout_passes=False),
        scratch_types=[
            pltpu.VMEM((T,), jnp.int32),
            pltpu.VMEM((NSUB, SUB), jnp.int32),
            pltpu.VMEM((NSUB, SUB), jnp.int32),
            pltpu.VMEM((SUB, D), jnp.float32),
            pltpu.VMEM((SUB, D), jnp.float32),
            pltpu.VMEM((SUB, D), jnp.float32),
            pltpu.VMEM((SUB,), jnp.float32),
            pltpu.VMEM((D,), jnp.float32),
            pltpu.VMEM((D,), jnp.float32),
            pltpu.VMEM((L,), jnp.int32),
            pltpu.VMEM((L,), jnp.int32),
            pltpu.SemaphoreType.DMA,
            pltpu.SemaphoreType.DMA,
            pltpu.SemaphoreType.DMA,
        ],
    )(_sc_body)
    return f(obs, act_i, rew_p, done_i, t0_vec, act_table, time_table, w, b)


def kernel(obs, act_p, rew_p, done, time_init, act_table, time_table, rew_W, rew_b):
    act_i = act_p.astype(jnp.int32)
    done_i = done.astype(jnp.int32)
    t0_vec = jnp.full((L,), time_init.astype(jnp.int32), jnp.int32)
    x, tv = _run(obs, act_i, rew_p, done_i, t0_vec, act_table, time_table,
                 rew_W.reshape(D), rew_b)
    return (x, done, tv[0])
